# baseline (speedup 1.0000x reference)
import functools

import jax
import jax.numpy as jnp
from jax import lax
from jax.experimental import pallas as pl
from jax.experimental.pallas import tpu as pltpu
from jax.experimental.pallas import tpu_sc as plsc

_NC = 2
_NS = 16
_L = 16
_NW = _NC * _NS


def _make_pool_kernel(B, D, half, denom):
    b_per_w = B // _NW
    CH = 64
    n_ch = b_per_w // CH
    NB = 8
    n_grp = 2 * CH // NB
    UN = 8
    nk = D // _L
    scale = jnp.float32(1.0 / denom)
    mesh = plsc.VectorSubcoreMesh(
        core_axis_name="c", subcore_axis_name="s",
        num_cores=_NC, num_subcores=_NS)

    @functools.partial(
        pl.kernel,
        mesh=mesh,
        compiler_params=pltpu.CompilerParams(use_tc_tiling_on_sc=False),
        out_type=jax.ShapeDtypeStruct((B, D), jnp.float32),
        scratch_types=[
            pltpu.VMEM((CH, 2, half), jnp.int32),
            [pltpu.VMEM((half, D), jnp.float32) for _ in range(NB)],
            pltpu.VMEM((CH, D), jnp.float32),
            [pltpu.SemaphoreType.DMA for _ in range(NB)],
        ],
    )
    def pool_k(x_hbm, emb_hbm, out_hbm, idx_v, bufs, pooled_v, sems):
        wid = lax.axis_index("s") * _NC + lax.axis_index("c")
        base = wid * b_per_w

        def accumulate(buf, init):
            def body(jj, accs):
                j0 = jj * UN
                new = list(accs)
                for u in range(UN):
                    for k in range(nk):
                        c = 2 * k + (u & 1)
                        new[c] = new[c] + buf[j0 + u, pl.ds(k * _L, _L)]
                return tuple(new)
            return lax.fori_loop(0, half // UN, body, init)

        zeros = tuple(jnp.zeros((_L,), jnp.float32) for _ in range(2 * nk))

        def chunk_body(c, carry):
            row0 = base + c * CH
            pltpu.sync_copy(x_hbm.at[pl.ds(row0, CH)], idx_v)
            for b in range(NB):
                pltpu.async_copy(
                    emb_hbm.at[idx_v.at[b // 2, b & 1]], bufs[b], sems[b])

            def grp_body(g, carry2):
                for b in range(NB):
                    r = g * (NB // 2) + (b // 2)
                    p = b & 1
                    pltpu.make_async_copy(
                        emb_hbm.at[idx_v.at[r, p]], bufs[b], sems[b]).wait()
                    if p == 0:
                        pair_accs = accumulate(bufs[b], zeros)
                    else:
                        accs = accumulate(bufs[b], pair_accs)
                        for k in range(nk):
                            pooled_v[r, pl.ds(k * _L, _L)] = (
                                accs[2 * k] + accs[2 * k + 1]) * scale

                    @pl.when(g < n_grp - 1)
                    def _issue():
                        pltpu.async_copy(
                            emb_hbm.at[idx_v.at[r + NB // 2, p]],
                            bufs[b], sems[b])
                return carry2

            lax.fori_loop(0, n_grp, grp_body, 0)
            pltpu.sync_copy(pooled_v, out_hbm.at[pl.ds(row0, CH)])
            return carry

        lax.fori_loop(0, n_ch, chunk_body, 0)

    return pool_k


def _mlp_block_kernel(p_ref, w1t_ref, b1_ref, w2t_ref, b2_ref, o_ref):
    h = jnp.dot(p_ref[...], w1t_ref[...], preferred_element_type=jnp.float32)
    h = jnp.maximum(h + b1_ref[...], 0.0)
    o = jnp.dot(h, w2t_ref[...], preferred_element_type=jnp.float32)
    o_ref[...] = o + b2_ref[...]


def _mlp(pooled, W1t, b1, W2t, b2, block_b=2048):
    B, D = pooled.shape
    N = W2t.shape[1]
    grid = (B // block_b,)
    return pl.pallas_call(
        _mlp_block_kernel,
        grid=grid,
        in_specs=[
            pl.BlockSpec((block_b, D), lambda i: (i, 0)),
            pl.BlockSpec((D, D), lambda i: (0, 0)),
            pl.BlockSpec((1, D), lambda i: (0, 0)),
            pl.BlockSpec((D, N), lambda i: (0, 0)),
            pl.BlockSpec((1, N), lambda i: (0, 0)),
        ],
        out_specs=pl.BlockSpec((block_b, N), lambda i: (i, 0)),
        out_shape=jax.ShapeDtypeStruct((B, N), jnp.float32),
    )(pooled, W1t, b1, W2t, b2)


def kernel(x, seq_lens, emb, W1, b1, W2, b2):
    del seq_lens
    B, H = x.shape
    V, D = emb.shape
    half = 104
    pad = 2 * half - H
    x = x.astype(jnp.int32)
    xp = jnp.concatenate(
        [x, jnp.full((B, pad), V - 1, jnp.int32)], axis=1).reshape(B, 2, half)
    pooled = _make_pool_kernel(B, D, half, float(H))(xp, emb)
    return _mlp(pooled, W1.T, b1.reshape(1, D), W2.T, b2.reshape(1, -1))


# speedup vs baseline: 3.5834x; 3.5834x over previous
import functools

import jax
import jax.numpy as jnp
from jax import lax
from jax.experimental import pallas as pl
from jax.experimental.pallas import tpu as pltpu
from jax.experimental.pallas import tpu_sc as plsc

_NC = 2
_NS = 16
_L = 16
_NW = _NC * _NS

_H0 = 104
_H1 = 96


def _make_pool_kernel(B, D, H, denom):
    b_per_w = B // _NW
    CH = 64
    n_ch = b_per_w // CH
    NB = 8
    n_grp = 2 * CH // NB
    UN = 8
    nk = D // _L
    scale = jnp.float32(1.0 / denom)
    mesh = plsc.VectorSubcoreMesh(
        core_axis_name="c", subcore_axis_name="s",
        num_cores=_NC, num_subcores=_NS)

    def half_idx(idx_v, r, p):
        if p == 0:
            return idx_v.at[r, pl.ds(0, _H0)]
        return idx_v.at[r, pl.ds(_H0, _H1)]

    def half_buf(buf, p):
        return buf.at[pl.ds(0, _H0)] if p == 0 else buf.at[pl.ds(0, _H1)]

    @functools.partial(
        pl.kernel,
        mesh=mesh,
        compiler_params=pltpu.CompilerParams(use_tc_tiling_on_sc=False),
        out_type=jax.ShapeDtypeStruct((B, D), jnp.float32),
        scratch_types=[
            pltpu.VMEM((CH, H), jnp.int32),
            [pltpu.VMEM((_H0, D), jnp.float32) for _ in range(NB)],
            pltpu.VMEM((CH, D), jnp.float32),
            [pltpu.SemaphoreType.DMA for _ in range(NB)],
        ],
    )
    def pool_k(x_hbm, emb_hbm, out_hbm, idx_v, bufs, pooled_v, sems):
        wid = lax.axis_index("s") * _NC + lax.axis_index("c")
        base = wid * b_per_w

        def accumulate(buf, nrows, init):
            def body(jj, accs):
                j0 = jj * UN
                new = list(accs)
                for u in range(UN):
                    for k in range(nk):
                        c = 2 * k + (u & 1)
                        new[c] = new[c] + buf[j0 + u, pl.ds(k * _L, _L)]
                return tuple(new)
            return lax.fori_loop(0, nrows // UN, body, init)

        zeros = tuple(jnp.zeros((_L,), jnp.float32) for _ in range(2 * nk))

        def chunk_body(c, carry):
            row0 = base + c * CH
            pltpu.sync_copy(x_hbm.at[pl.ds(row0, CH)], idx_v)
            for b in range(NB):
                pltpu.async_copy(
                    emb_hbm.at[half_idx(idx_v, b // 2, b & 1)],
                    half_buf(bufs[b], b & 1), sems[b])

            def grp_body(g, carry2):
                for b in range(NB):
                    r = g * (NB // 2) + (b // 2)
                    p = b & 1
                    pltpu.make_async_copy(
                        emb_hbm.at[half_idx(idx_v, r, p)],
                        half_buf(bufs[b], p), sems[b]).wait()
                    if p == 0:
                        pair_accs = accumulate(bufs[b], _H0, zeros)
                    else:
                        accs = accumulate(bufs[b], _H1, pair_accs)
                        for k in range(nk):
                            pooled_v[r, pl.ds(k * _L, _L)] = (
                                accs[2 * k] + accs[2 * k + 1]) * scale

                    @pl.when(g < n_grp - 1)
                    def _issue():
                        pltpu.async_copy(
                            emb_hbm.at[half_idx(idx_v, r + NB // 2, p)],
                            half_buf(bufs[b], p), sems[b])
                return carry2

            lax.fori_loop(0, n_grp, grp_body, 0)
            pltpu.sync_copy(pooled_v, out_hbm.at[pl.ds(row0, CH)])
            return carry

        lax.fori_loop(0, n_ch, chunk_body, 0)

    return pool_k


def _mlp_block_kernel(p_ref, w1t_ref, b1_ref, w2t_ref, b2_ref, o_ref):
    h = jnp.dot(p_ref[...], w1t_ref[...], preferred_element_type=jnp.float32)
    h = jnp.maximum(h + b1_ref[...], 0.0)
    o = jnp.dot(h, w2t_ref[...], preferred_element_type=jnp.float32)
    o_ref[...] = o + b2_ref[...]


def _mlp(pooled, W1t, b1, W2t, b2, block_b=2048):
    B, D = pooled.shape
    N = W2t.shape[1]
    grid = (B // block_b,)
    return pl.pallas_call(
        _mlp_block_kernel,
        grid=grid,
        in_specs=[
            pl.BlockSpec((block_b, D), lambda i: (i, 0)),
            pl.BlockSpec((D, D), lambda i: (0, 0)),
            pl.BlockSpec((1, D), lambda i: (0, 0)),
            pl.BlockSpec((D, N), lambda i: (0, 0)),
            pl.BlockSpec((1, N), lambda i: (0, 0)),
        ],
        out_specs=pl.BlockSpec((block_b, N), lambda i: (i, 0)),
        out_shape=jax.ShapeDtypeStruct((B, N), jnp.float32),
    )(pooled, W1t, b1, W2t, b2)


def kernel(x, seq_lens, emb, W1, b1, W2, b2):
    del seq_lens
    B, H = x.shape
    V, D = emb.shape
    x = x.astype(jnp.int32)
    pooled = _make_pool_kernel(B, D, H, float(H))(x, emb)
    return _mlp(pooled, W1.T, b1.reshape(1, D), W2.T, b2.reshape(1, -1))
